# baseline (speedup 1.0000x reference)
import jax
import jax.numpy as jnp
from jax.experimental import pallas as pl
from jax.experimental.pallas import tpu as pltpu

_D0, _D1, _K = 2048, 2048, 4
_W = _D1 * _K
_BR = 128
_BL = 4 * _BR
_NR = _D0 // _BR
_NC = _W // _BL


def _mask_block_kernel(x_ref, o_ref):
    r = pl.program_id(0)
    c = pl.program_id(1)

    @pl.when(c < r)
    def _zero():
        o_ref[...] = jnp.zeros_like(o_ref)

    @pl.when(c > r)
    def _copy():
        o_ref[...] = x_ref[...]

    @pl.when(c == r)
    def _diag():
        rows = jax.lax.broadcasted_iota(jnp.int32, (_BR, _BL), 0)
        lanes = jax.lax.broadcasted_iota(jnp.int32, (_BR, _BL), 1)
        keep = lanes >= 4 * rows
        o_ref[...] = jnp.where(keep, x_ref[...], 0.0)


def _in_index(r, c):
    is_zero = c < r
    return (jnp.where(is_zero, r - 1, r), jnp.where(is_zero, _NC - 1, c))


def kernel(w):
    x = w.reshape(_D0, _W)
    out = pl.pallas_call(
        _mask_block_kernel,
        grid=(_NR, _NC),
        in_specs=[pl.BlockSpec((_BR, _BL), _in_index)],
        out_specs=pl.BlockSpec((_BR, _BL), lambda r, c: (r, c)),
        out_shape=jax.ShapeDtypeStruct((_D0, _W), jnp.float32),
    )(x)
    return out.reshape(w.shape)


# speedup vs baseline: 1.2228x; 1.2228x over previous
import jax
import jax.numpy as jnp
from jax.experimental import pallas as pl
from jax.experimental.pallas import tpu as pltpu

_D0, _D1, _K = 2048, 2048, 4
_W = _D1 * _K
_BR = 256
_NR = _D0 // _BR


def _mask_kernel(x_ref, o_ref):
    r = pl.program_id(0)
    rows = jax.lax.broadcasted_iota(jnp.int32, (_BR, _W), 0) + r * _BR
    lanes = jax.lax.broadcasted_iota(jnp.int32, (_BR, _W), 1)
    keep = lanes >= 4 * rows
    o_ref[...] = jnp.where(keep, x_ref[...], 0.0)


def kernel(w):
    x = w.reshape(_D0, _W)
    out = pl.pallas_call(
        _mask_kernel,
        grid=(_NR,),
        in_specs=[pl.BlockSpec((_BR, _W), lambda r: (r, 0))],
        out_specs=pl.BlockSpec((_BR, _W), lambda r: (r, 0)),
        out_shape=jax.ShapeDtypeStruct((_D0, _W), jnp.float32),
    )(x)
    return out.reshape(w.shape)
